# Initial kernel scaffold; baseline (speedup 1.0000x reference)
import jax
import jax.numpy as jnp
from jax.experimental import pallas as pl


def kernel(feature, adj, weight, bias, cat_w, cat_b):
    raise NotImplementedError("write your pallas kernel here")


# speedup vs baseline: 1.0396x; 1.0396x over previous
import functools

import jax
import jax.numpy as jnp
from jax.experimental import pallas as pl
from jax.experimental.pallas import tpu as pltpu

N = 4096
FEAT = 128
EMB = 128
ALPHA = 0.2
BLK = 256


def _fused_kernel(feature_ref, adj_ref, weight_ref, cat_w_ref, bias_ref,
                  cat_b_ref, out_ref, s_ref, t_ref):
    p = pl.program_id(0)
    i = pl.program_id(1)

    @pl.when(jnp.logical_and(p == 0, i == 0))
    def _compute_support():
        s = jnp.dot(feature_ref[...], weight_ref[...],
                    preferred_element_type=jnp.float32)
        s_ref[...] = jnp.maximum(s, 0.0)

    @pl.when(p == 0)
    def _first_hop():
        t_ref[pl.ds(i * BLK, BLK), :] = jnp.dot(
            adj_ref[...], s_ref[...], preferred_element_type=jnp.float32)

    @pl.when(p == 1)
    def _second_hop_and_epilogue():
        m = jnp.dot(adj_ref[...], t_ref[...],
                    preferred_element_type=jnp.float32)
        rows = pl.ds(i * BLK, BLK)
        s_blk = s_ref[rows, :]
        low = t_ref[rows, :] + s_blk
        mid = m - s_blk
        contract = (((1,), (1,)), ((), ()))
        lin = jax.lax.dot_general(low, cat_w_ref[:, :EMB], contract,
                                  preferred_element_type=jnp.float32)
        lin += jax.lax.dot_general(mid, cat_w_ref[:, EMB:], contract,
                                   preferred_element_type=jnp.float32)
        lin += cat_b_ref[...]
        out_ref[...] = jnp.where(lin >= 0, lin, ALPHA * lin) + bias_ref[...]


@functools.partial(jax.jit, static_argnames=())
def kernel(feature, adj, weight, bias, cat_w, cat_b):
    nb = N // BLK
    full = lambda shape: pl.BlockSpec(shape, lambda p, i: (0, 0))
    out = pl.pallas_call(
        _fused_kernel,
        grid=(2, nb),
        in_specs=[
            full((N, FEAT)),
            pl.BlockSpec((BLK, N), lambda p, i: (i, 0)),
            full((FEAT, EMB)),
            full((EMB, 2 * EMB)),
            full((1, EMB)),
            full((1, EMB)),
        ],
        out_specs=pl.BlockSpec((BLK, EMB), lambda p, i: (i * p, 0)),
        out_shape=jax.ShapeDtypeStruct((N, EMB), jnp.float32),
        scratch_shapes=[
            pltpu.VMEM((N, EMB), jnp.float32),
            pltpu.VMEM((N, EMB), jnp.float32),
        ],
    )(feature, adj, weight, cat_w,
      bias.reshape(1, EMB), cat_b.reshape(1, EMB))
    return out


# speedup vs baseline: 1.2672x; 1.2189x over previous
import functools

import jax
import jax.numpy as jnp
from jax.experimental import pallas as pl
from jax.experimental.pallas import tpu as pltpu

N = 4096
FEAT = 128
EMB = 128
ALPHA = 0.2
BLK = 512


def _fused_kernel(feature_ref, adj_ref, weight_ref, cat_w_ref, bias_ref,
                  cat_b_ref, out_ref, s_ref, t_ref):
    p = pl.program_id(0)
    i = pl.program_id(1)

    @pl.when(jnp.logical_and(p == 0, i == 0))
    def _compute_support():
        s = jnp.dot(feature_ref[...], weight_ref[...],
                    preferred_element_type=jnp.float32)
        s_ref[...] = jnp.maximum(s, 0.0)

    @pl.when(p == 0)
    def _first_hop():
        t_ref[pl.ds(i * BLK, BLK), :] = jnp.dot(
            adj_ref[...], s_ref[...], preferred_element_type=jnp.float32)

    @pl.when(p == 1)
    def _second_hop_and_epilogue():
        m = jnp.dot(adj_ref[...], t_ref[...],
                    preferred_element_type=jnp.float32)
        rows = pl.ds(i * BLK, BLK)
        s_blk = s_ref[rows, :]
        low = t_ref[rows, :] + s_blk
        mid = m - s_blk
        contract = (((1,), (1,)), ((), ()))
        lin = jax.lax.dot_general(low, cat_w_ref[:, :EMB], contract,
                                  preferred_element_type=jnp.float32)
        lin += jax.lax.dot_general(mid, cat_w_ref[:, EMB:], contract,
                                   preferred_element_type=jnp.float32)
        lin += cat_b_ref[...]
        out_ref[...] = jnp.where(lin >= 0, lin, ALPHA * lin) + bias_ref[...]


@functools.partial(jax.jit, static_argnames=())
def kernel(feature, adj, weight, bias, cat_w, cat_b):
    nb = N // BLK
    full = lambda shape: pl.BlockSpec(shape, lambda p, i: (0, 0))
    out = pl.pallas_call(
        _fused_kernel,
        grid=(2, nb),
        in_specs=[
            full((N, FEAT)),
            pl.BlockSpec((BLK, N), lambda p, i: (i, 0)),
            full((FEAT, EMB)),
            full((EMB, 2 * EMB)),
            full((1, EMB)),
            full((1, EMB)),
        ],
        out_specs=pl.BlockSpec((BLK, EMB), lambda p, i: (i * p, 0)),
        out_shape=jax.ShapeDtypeStruct((N, EMB), jnp.float32),
        scratch_shapes=[
            pltpu.VMEM((N, EMB), jnp.float32),
            pltpu.VMEM((N, EMB), jnp.float32),
        ],
    )(feature, adj, weight, cat_w,
      bias.reshape(1, EMB), cat_b.reshape(1, EMB))
    return out
